# Initial kernel scaffold; baseline (speedup 1.0000x reference)
import jax
import jax.numpy as jnp
from jax.experimental import pallas as pl


def kernel(image_features, text_features):
    raise NotImplementedError("write your pallas kernel here")


# speedup vs baseline: 9.3399x; 9.3399x over previous
import functools

import jax
import jax.numpy as jnp
from jax.experimental import pallas as pl

_N = 8192
_D = 512
_K = 64
_ITERS = 10
_MAXF = 32


def _selector_body(x_ref, t_ref, sel_ref, f2t_ref):
    x = x_ref[...]
    t = t_ref[...]

    x2 = jnp.sum(x * x, axis=1, keepdims=True)
    xn = x / jnp.clip(jnp.sqrt(x2), 1e-8)
    tn = t / jnp.clip(jnp.sqrt(jnp.sum(t * t)), 1e-8)
    f2t = jnp.sum(xn * tn, axis=1)

    k_iota = jax.lax.broadcasted_iota(jnp.int32, (_N, _K), 1)
    n_iota = jax.lax.broadcasted_iota(jnp.int32, (_N, _K), 0)

    def _labels(c):
        c2 = jnp.sum(c * c, axis=1)
        d2 = x2 - 2.0 * jnp.dot(x, c.T) + c2[None, :]
        dmin = jnp.min(d2, axis=1, keepdims=True)
        return jnp.min(jnp.where(d2 == dmin, k_iota, _K), axis=1)

    def _step(_, c):
        labels = _labels(c)
        oh = (labels[:, None] == k_iota[:1, :]).astype(jnp.float32)
        sums = jax.lax.dot_general(
            oh, x, (((0,), (0,)), ((), ())),
            precision=jax.lax.Precision.HIGHEST)
        counts = jnp.sum(oh, axis=0)
        return jnp.where(counts[:, None] > 0,
                         sums / jnp.clip(counts[:, None], 1.0, None), c)

    c = jax.lax.fori_loop(0, _ITERS, _step, x[:_K, :])
    labels = _labels(c)

    masked = jnp.where(labels[:, None] == k_iota[:1, :],
                       f2t[:, None], -1e9)
    top_score = jnp.max(masked, axis=0)
    top_idx = jnp.min(
        jnp.where(masked == top_score[None, :], n_iota, _N), axis=0)

    s_col = top_score[:, None]
    s_row = top_score[None, :]
    i_iota = jax.lax.broadcasted_iota(jnp.int32, (_K, _K), 0)
    j_iota = jax.lax.broadcasted_iota(jnp.int32, (_K, _K), 1)
    before = (s_row > s_col) | ((s_row == s_col) & (j_iota < i_iota))
    rank = jnp.sum(before.astype(jnp.int32), axis=1)
    selected = (rank < _MAXF) & (top_score > -1e8)

    hit = (n_iota == top_idx[None, :]) & selected[None, :]
    sel_ref[...] = jnp.max(hit.astype(jnp.int32), axis=1)
    f2t_ref[...] = f2t


@functools.partial(jax.jit, static_argnames=())
def _run(image_features, text_features):
    return pl.pallas_call(
        _selector_body,
        out_shape=(
            jax.ShapeDtypeStruct((_N,), jnp.int32),
            jax.ShapeDtypeStruct((_N,), jnp.float32),
        ),
    )(image_features, text_features)


def kernel(image_features, text_features):
    is_selected, f2t = _run(image_features, text_features)
    return is_selected, f2t, image_features
